# baseline (speedup 1.0000x reference)
import functools

import jax
import jax.numpy as jnp
from jax import lax
from jax.experimental import pallas as pl
from jax.experimental.pallas import tpu as pltpu
from jax.experimental.pallas import tpu_sc as plsc

N = 10000
NPAD = 10240
E = 320000
D = 128
ROWW = D + 16
NC, NS = 2, 16
NW = NC * NS
CH = 256
NCHUNKS = E // CH
SLOTS = 40
ZPT = NPAD // NS
DPT = N // NS


def _sc_body(xpad_hbm, ei_hbm, zeros_hbm, out_hbm,
             eidx0, eidx1, rows, acc,
             si0, si1, sg0, sz):
    c = lax.axis_index("c")
    s = lax.axis_index("s")
    wid = c * NS + s

    zslice = acc.at[pl.ds(pl.multiple_of(s * ZPT, 8), ZPT)]
    pltpu.async_copy(zeros_hbm, zslice, sz)

    def off(t):
        return jnp.minimum(wid + t * NW, NCHUNKS - 1) * CH

    def idx_start(t, ebuf, sem):
        pltpu.async_copy(ei_hbm.at[:, pl.ds(off(t), CH)], ebuf, sem)

    def idx_wait(ebuf, sem):
        pltpu.make_async_copy(ei_hbm.at[:, pl.ds(0, CH)], ebuf, sem).wait()

    def gath(sbuf, rbuf, sem):
        pltpu.async_copy(xpad_hbm.at[sbuf], rbuf, sem)

    def gwait(sbuf, rbuf, sem):
        pltpu.make_async_copy(xpad_hbm.at[sbuf], rbuf, sem).wait()

    def scat(t, ebuf):
        @pl.when(wid + t * NW < NCHUNKS)
        def _():
            pltpu.sync_copy(rows, acc.at[ebuf.at[1]], add=True)

    idx_start(0, eidx0, si0)
    idx_start(1, eidx1, si1)
    idx_wait(eidx0, si0)
    gath(eidx0.at[0], rows, sg0)
    pltpu.make_async_copy(zeros_hbm, zslice, sz).wait()
    plsc.subcore_barrier()

    def body(i, carry):
        a = 2 * i
        b = a + 1
        gwait(eidx0.at[0], rows, sg0)
        scat(a, eidx0)
        idx_start(a + 2, eidx0, si0)
        idx_wait(eidx1, si1)
        gath(eidx1.at[0], rows, sg0)
        gwait(eidx1.at[0], rows, sg0)
        scat(b, eidx1)
        idx_start(b + 2, eidx1, si1)
        idx_wait(eidx0, si0)
        gath(eidx0.at[0], rows, sg0)
        return carry

    lax.fori_loop(0, SLOTS // 2, body, 0)
    idx_wait(eidx1, si1)
    gwait(eidx0.at[0], rows, sg0)

    plsc.subcore_barrier()
    pltpu.sync_copy(acc.at[pl.ds(s * DPT, DPT)],
                    out_hbm.at[pl.ds(c * N + s * DPT, DPT)])


_sc_aggregate = functools.partial(
    pl.kernel,
    out_type=jax.ShapeDtypeStruct((NC * N, ROWW), jnp.float32),
    mesh=plsc.VectorSubcoreMesh(core_axis_name="c", subcore_axis_name="s",
                                num_cores=NC, num_subcores=NS),
    scratch_types=[
        pltpu.VMEM((2, CH), jnp.int32),
        pltpu.VMEM((2, CH), jnp.int32),
        pltpu.VMEM((CH, ROWW), jnp.float32),
        pltpu.VMEM_SHARED((NPAD, ROWW), jnp.float32),
        pltpu.SemaphoreType.DMA,
        pltpu.SemaphoreType.DMA,
        pltpu.SemaphoreType.DMA,
        pltpu.SemaphoreType.DMA,
    ],
    compiler_params=pltpu.CompilerParams(use_tc_tiling_on_sc=False),
)(_sc_body)


DN = (((1,), (1,)), ((), ()))


def _tc_pre_body(x_ref, wr_ref, b_ref, o_ref):
    o_ref[...] = lax.dot_general(
        x_ref[...], wr_ref[...], DN,
        preferred_element_type=jnp.float32) + b_ref[...]


def _tc_pre(x, w_r, b2):
    R = 1000
    return pl.pallas_call(
        _tc_pre_body,
        grid=(N // R,),
        in_specs=[
            pl.BlockSpec((R, D), lambda i: (i, 0)),
            pl.BlockSpec((D, D), lambda i: (0, 0)),
            pl.BlockSpec((1, D), lambda i: (0, 0)),
        ],
        out_specs=pl.BlockSpec((R, D), lambda i: (i, 0)),
        out_shape=jax.ShapeDtypeStruct((N, D), jnp.float32),
    )(x, w_r, b2)


def _tc_body(p_ref, xr_ref, wl_ref, o_ref):
    p = p_ref[0] + p_ref[1]
    cnt = p[:, D:D + 1]
    mean = p[:, :D] / jnp.maximum(cnt, 1.0)
    o_ref[...] = lax.dot_general(
        mean, wl_ref[...], DN,
        preferred_element_type=jnp.float32) + xr_ref[...]


def _tc_combine(partial, xr, w_l):
    R = 1000
    return pl.pallas_call(
        _tc_body,
        grid=(N // R,),
        in_specs=[
            pl.BlockSpec((NC, R, ROWW), lambda i: (0, i, 0)),
            pl.BlockSpec((R, D), lambda i: (i, 0)),
            pl.BlockSpec((D, D), lambda i: (0, 0)),
        ],
        out_specs=pl.BlockSpec((R, D), lambda i: (i, 0)),
        out_shape=jax.ShapeDtypeStruct((N, D), jnp.float32),
    )(partial, xr, w_l)


def kernel(x, edge_index, W_l, b_l, W_r):
    xpad = jnp.concatenate([x, jnp.ones((N, ROWW - D), jnp.float32)], axis=1)
    zeros = jnp.zeros((ZPT, ROWW), jnp.float32)
    xr = _tc_pre(x, W_r, b_l.reshape(1, D))
    partial = _sc_aggregate(xpad, edge_index, zeros)
    partial = partial.reshape(NC, N, ROWW)
    return _tc_combine(partial, xr, W_l)


# speedup vs baseline: 1.0191x; 1.0191x over previous
import functools

import jax
import jax.numpy as jnp
from jax import lax
from jax.experimental import pallas as pl
from jax.experimental.pallas import tpu as pltpu
from jax.experimental.pallas import tpu_sc as plsc

N = 10000
NPAD = 10240
E = 320000
D = 128
ROWW = D + 16
NC, NS = 2, 16
NW = NC * NS
CH = 256
NCHUNKS = E // CH
SLOTS = 40
ZPT = NPAD // NS
DPT = N // NS


def _sc_body(xpad_hbm, ei_hbm, zeros_hbm, out_hbm,
             eidx0, eidx1, rows, acc,
             si0, si1, sg0, sz):
    c = lax.axis_index("c")
    s = lax.axis_index("s")
    wid = c * NS + s

    zslice = acc.at[pl.ds(pl.multiple_of(s * ZPT, 8), ZPT)]
    pltpu.async_copy(zeros_hbm, zslice, sz)

    def off(t):
        return jnp.minimum(wid + t * NW, NCHUNKS - 1) * CH

    def idx_start(t, ebuf, sem):
        pltpu.async_copy(ei_hbm.at[:, pl.ds(off(t), CH)], ebuf, sem)

    def idx_wait(ebuf, sem):
        pltpu.make_async_copy(ei_hbm.at[:, pl.ds(0, CH)], ebuf, sem).wait()

    def gath(sbuf, rbuf, sem):
        pltpu.async_copy(xpad_hbm.at[sbuf], rbuf, sem)

    def gwait(sbuf, rbuf, sem):
        pltpu.make_async_copy(xpad_hbm.at[sbuf], rbuf, sem).wait()

    def scat(t, ebuf):
        @pl.when(wid + t * NW < NCHUNKS)
        def _():
            pltpu.sync_copy(rows, acc.at[ebuf.at[1]], add=True)

    idx_start(0, eidx0, si0)
    idx_start(1, eidx1, si1)
    idx_wait(eidx0, si0)
    gath(eidx0.at[0], rows, sg0)
    pltpu.make_async_copy(zeros_hbm, zslice, sz).wait()
    plsc.subcore_barrier()

    def body(i, carry):
        a = 2 * i
        b = a + 1
        gwait(eidx0.at[0], rows, sg0)
        scat(a, eidx0)
        idx_start(a + 2, eidx0, si0)
        idx_wait(eidx1, si1)
        gath(eidx1.at[0], rows, sg0)
        gwait(eidx1.at[0], rows, sg0)
        scat(b, eidx1)
        idx_start(b + 2, eidx1, si1)
        idx_wait(eidx0, si0)
        gath(eidx0.at[0], rows, sg0)
        return carry

    lax.fori_loop(0, SLOTS // 2, body, 0)
    idx_wait(eidx1, si1)
    gwait(eidx0.at[0], rows, sg0)

    plsc.subcore_barrier()
    pltpu.sync_copy(acc.at[pl.ds(s * DPT, DPT)],
                    out_hbm.at[pl.ds(c * N + s * DPT, DPT)])


_sc_aggregate = functools.partial(
    pl.kernel,
    out_type=jax.ShapeDtypeStruct((NC * N, ROWW), jnp.float32),
    mesh=plsc.VectorSubcoreMesh(core_axis_name="c", subcore_axis_name="s",
                                num_cores=NC, num_subcores=NS),
    scratch_types=[
        pltpu.VMEM((2, CH), jnp.int32),
        pltpu.VMEM((2, CH), jnp.int32),
        pltpu.VMEM((CH, ROWW), jnp.float32),
        pltpu.VMEM_SHARED((NPAD, ROWW), jnp.float32),
        pltpu.SemaphoreType.DMA,
        pltpu.SemaphoreType.DMA,
        pltpu.SemaphoreType.DMA,
        pltpu.SemaphoreType.DMA,
    ],
    compiler_params=pltpu.CompilerParams(use_tc_tiling_on_sc=False),
)(_sc_body)


def _tc_body(p_ref, x_ref, wl_ref, wr_ref, b_ref, o_ref):
    p = p_ref[0] + p_ref[1]
    cnt = p[:, D:D + 1]
    mean = p[:, :D] / jnp.maximum(cnt, 1.0)
    dn = (((1,), (1,)), ((), ()))
    o_ref[...] = (
        lax.dot_general(mean, wl_ref[...], dn,
                        preferred_element_type=jnp.float32)
        + lax.dot_general(x_ref[...], wr_ref[...], dn,
                          preferred_element_type=jnp.float32)
        + b_ref[...]
    )


def _tc_combine(partial, x, wl_t, wr_t, b2):
    R = 2000
    grid = (N // R,)
    return pl.pallas_call(
        _tc_body,
        grid=grid,
        in_specs=[
            pl.BlockSpec((NC, R, ROWW), lambda i: (0, i, 0)),
            pl.BlockSpec((R, D), lambda i: (i, 0)),
            pl.BlockSpec((D, D), lambda i: (0, 0)),
            pl.BlockSpec((D, D), lambda i: (0, 0)),
            pl.BlockSpec((1, D), lambda i: (0, 0)),
        ],
        out_specs=pl.BlockSpec((R, D), lambda i: (i, 0)),
        out_shape=jax.ShapeDtypeStruct((N, D), jnp.float32),
    )(partial, x, wl_t, wr_t, b2)


def kernel(x, edge_index, W_l, b_l, W_r):
    xpad = jnp.concatenate([x, jnp.ones((N, ROWW - D), jnp.float32)], axis=1)
    zeros = jnp.zeros((ZPT, ROWW), jnp.float32)
    partial = _sc_aggregate(xpad, edge_index, zeros)
    partial = partial.reshape(NC, N, ROWW)
    return _tc_combine(partial, x, W_l, W_r, b_l.reshape(1, D))


# speedup vs baseline: 1.0313x; 1.0121x over previous
import functools

import jax
import jax.numpy as jnp
from jax import lax
from jax.experimental import pallas as pl
from jax.experimental.pallas import tpu as pltpu
from jax.experimental.pallas import tpu_sc as plsc

N = 10000
NPAD = 10240
E = 320000
D = 128
ROWW = D + 16
NC, NS = 2, 16
NW = NC * NS
CH = 256
NCHUNKS = E // CH
SLOTS = 40
ZPT = NPAD // NS
DPT = N // NS


def _sc_body(xpad_hbm, ei_hbm, zeros_hbm, out_hbm,
             eidx0, eidx1, rows, acc,
             si0, si1, sg0, sg1, sz):
    c = lax.axis_index("c")
    s = lax.axis_index("s")
    wid = c * NS + s

    zslice = acc.at[pl.ds(pl.multiple_of(s * ZPT, 8), ZPT)]
    pltpu.async_copy(zeros_hbm, zslice, sz)

    def off(t):
        return jnp.minimum(wid + t * NW, NCHUNKS - 1) * CH

    def idx_start(t, ebuf, sem):
        pltpu.async_copy(ei_hbm.at[:, pl.ds(off(t), CH)], ebuf, sem)

    def idx_wait(ebuf, sem):
        pltpu.make_async_copy(ei_hbm.at[:, pl.ds(0, CH)], ebuf, sem).wait()

    H = CH // 2

    def gath(ebuf):
        pltpu.async_copy(xpad_hbm.at[ebuf.at[0, pl.ds(0, H)]],
                         rows.at[pl.ds(0, H)], sg0)
        pltpu.async_copy(xpad_hbm.at[ebuf.at[0, pl.ds(H, H)]],
                         rows.at[pl.ds(H, H)], sg1)

    def gwait(ebuf):
        pltpu.make_async_copy(xpad_hbm.at[ebuf.at[0, pl.ds(0, H)]],
                              rows.at[pl.ds(0, H)], sg0).wait()
        pltpu.make_async_copy(xpad_hbm.at[ebuf.at[0, pl.ds(H, H)]],
                              rows.at[pl.ds(H, H)], sg1).wait()

    def scat(t, ebuf):
        @pl.when(wid + t * NW < NCHUNKS)
        def _():
            pltpu.sync_copy(rows, acc.at[ebuf.at[1]], add=True)

    idx_start(0, eidx0, si0)
    idx_start(1, eidx1, si1)
    idx_wait(eidx0, si0)
    gath(eidx0)
    pltpu.make_async_copy(zeros_hbm, zslice, sz).wait()
    plsc.subcore_barrier()

    def body(i, carry):
        a = 2 * i
        b = a + 1
        gwait(eidx0)
        scat(a, eidx0)
        idx_start(a + 2, eidx0, si0)
        idx_wait(eidx1, si1)
        gath(eidx1)
        gwait(eidx1)
        scat(b, eidx1)
        idx_start(b + 2, eidx1, si1)
        idx_wait(eidx0, si0)
        gath(eidx0)
        return carry

    lax.fori_loop(0, SLOTS // 2, body, 0)
    idx_wait(eidx1, si1)
    gwait(eidx0)

    plsc.subcore_barrier()
    pltpu.sync_copy(acc.at[pl.ds(s * DPT, DPT)],
                    out_hbm.at[pl.ds(c * N + s * DPT, DPT)])


_sc_aggregate = functools.partial(
    pl.kernel,
    out_type=jax.ShapeDtypeStruct((NC * N, ROWW), jnp.float32),
    mesh=plsc.VectorSubcoreMesh(core_axis_name="c", subcore_axis_name="s",
                                num_cores=NC, num_subcores=NS),
    scratch_types=[
        pltpu.VMEM((2, CH), jnp.int32),
        pltpu.VMEM((2, CH), jnp.int32),
        pltpu.VMEM((CH, ROWW), jnp.float32),
        pltpu.VMEM_SHARED((NPAD, ROWW), jnp.float32),
        pltpu.SemaphoreType.DMA,
        pltpu.SemaphoreType.DMA,
        pltpu.SemaphoreType.DMA,
        pltpu.SemaphoreType.DMA,
        pltpu.SemaphoreType.DMA,
    ],
    compiler_params=pltpu.CompilerParams(use_tc_tiling_on_sc=False),
)(_sc_body)


def _tc_body(p_ref, x_ref, wl_ref, wr_ref, b_ref, o_ref):
    p = p_ref[0] + p_ref[1]
    cnt = p[:, D:D + 1]
    mean = p[:, :D] / jnp.maximum(cnt, 1.0)
    dn = (((1,), (1,)), ((), ()))
    o_ref[...] = (
        lax.dot_general(mean, wl_ref[...], dn,
                        preferred_element_type=jnp.float32)
        + lax.dot_general(x_ref[...], wr_ref[...], dn,
                          preferred_element_type=jnp.float32)
        + b_ref[...]
    )


def _tc_combine(partial, x, wl_t, wr_t, b2):
    R = 2000
    grid = (N // R,)
    return pl.pallas_call(
        _tc_body,
        grid=grid,
        in_specs=[
            pl.BlockSpec((NC, R, ROWW), lambda i: (0, i, 0)),
            pl.BlockSpec((R, D), lambda i: (i, 0)),
            pl.BlockSpec((D, D), lambda i: (0, 0)),
            pl.BlockSpec((D, D), lambda i: (0, 0)),
            pl.BlockSpec((1, D), lambda i: (0, 0)),
        ],
        out_specs=pl.BlockSpec((R, D), lambda i: (i, 0)),
        out_shape=jax.ShapeDtypeStruct((N, D), jnp.float32),
    )(partial, x, wl_t, wr_t, b2)


def kernel(x, edge_index, W_l, b_l, W_r):
    xpad = jnp.concatenate([x, jnp.ones((N, ROWW - D), jnp.float32)], axis=1)
    zeros = jnp.zeros((ZPT, ROWW), jnp.float32)
    partial = _sc_aggregate(xpad, edge_index, zeros)
    partial = partial.reshape(NC, N, ROWW)
    return _tc_combine(partial, x, W_l, W_r, b_l.reshape(1, D))


# speedup vs baseline: 1.0328x; 1.0014x over previous
import functools

import jax
import jax.numpy as jnp
from jax import lax
from jax.experimental import pallas as pl
from jax.experimental.pallas import tpu as pltpu
from jax.experimental.pallas import tpu_sc as plsc

N = 10000
NPAD = 10240
E = 320000
D = 128
ROWW = D + 16
NC, NS = 2, 16
NW = NC * NS
CH = 256
NCHUNKS = E // CH
SLOTS = 40
ZPT = NPAD // NS
DPT = N // NS


def _sc_body(xpad_hbm, ei_hbm, zeros_hbm, out_hbm,
             eidx0, eidx1, rows, acc,
             si0, si1, sg0, sg1, ss0, ss1, sz):
    c = lax.axis_index("c")
    s = lax.axis_index("s")
    wid = c * NS + s

    zslice = acc.at[pl.ds(pl.multiple_of(s * ZPT, 8), ZPT)]
    pltpu.async_copy(zeros_hbm, zslice, sz)

    def off(t):
        return jnp.minimum(wid + t * NW, NCHUNKS - 1) * CH

    def idx_start(t, ebuf, sem):
        pltpu.async_copy(ei_hbm.at[:, pl.ds(off(t), CH)], ebuf, sem)

    def idx_wait(ebuf, sem):
        pltpu.make_async_copy(ei_hbm.at[:, pl.ds(0, CH)], ebuf, sem).wait()

    H = CH // 2

    def gath(ebuf):
        pltpu.async_copy(xpad_hbm.at[ebuf.at[0, pl.ds(0, H)]],
                         rows.at[pl.ds(0, H)], sg0)
        pltpu.async_copy(xpad_hbm.at[ebuf.at[0, pl.ds(H, H)]],
                         rows.at[pl.ds(H, H)], sg1)

    def gwait(ebuf):
        pltpu.make_async_copy(xpad_hbm.at[ebuf.at[0, pl.ds(0, H)]],
                              rows.at[pl.ds(0, H)], sg0).wait()
        pltpu.make_async_copy(xpad_hbm.at[ebuf.at[0, pl.ds(H, H)]],
                              rows.at[pl.ds(H, H)], sg1).wait()

    def scat(t, ebuf):
        @pl.when(wid + t * NW < NCHUNKS)
        def _():
            pltpu.async_copy(rows.at[pl.ds(0, H)],
                             acc.at[ebuf.at[1, pl.ds(0, H)]], ss0, add=True)
            pltpu.async_copy(rows.at[pl.ds(H, H)],
                             acc.at[ebuf.at[1, pl.ds(H, H)]], ss1, add=True)
            pltpu.make_async_copy(rows.at[pl.ds(0, H)],
                                  acc.at[ebuf.at[1, pl.ds(0, H)]], ss0).wait()
            pltpu.make_async_copy(rows.at[pl.ds(H, H)],
                                  acc.at[ebuf.at[1, pl.ds(H, H)]], ss1).wait()

    idx_start(0, eidx0, si0)
    idx_start(1, eidx1, si1)
    idx_wait(eidx0, si0)
    gath(eidx0)
    pltpu.make_async_copy(zeros_hbm, zslice, sz).wait()
    plsc.subcore_barrier()

    def body(i, carry):
        a = 2 * i
        b = a + 1
        gwait(eidx0)
        scat(a, eidx0)
        idx_start(a + 2, eidx0, si0)
        idx_wait(eidx1, si1)
        gath(eidx1)
        gwait(eidx1)
        scat(b, eidx1)
        idx_start(b + 2, eidx1, si1)
        idx_wait(eidx0, si0)
        gath(eidx0)
        return carry

    lax.fori_loop(0, SLOTS // 2, body, 0)
    idx_wait(eidx1, si1)
    gwait(eidx0)

    plsc.subcore_barrier()
    pltpu.sync_copy(acc.at[pl.ds(s * DPT, DPT)],
                    out_hbm.at[pl.ds(c * N + s * DPT, DPT)])


_sc_aggregate = functools.partial(
    pl.kernel,
    out_type=jax.ShapeDtypeStruct((NC * N, ROWW), jnp.float32),
    mesh=plsc.VectorSubcoreMesh(core_axis_name="c", subcore_axis_name="s",
                                num_cores=NC, num_subcores=NS),
    scratch_types=[
        pltpu.VMEM((2, CH), jnp.int32),
        pltpu.VMEM((2, CH), jnp.int32),
        pltpu.VMEM((CH, ROWW), jnp.float32),
        pltpu.VMEM_SHARED((NPAD, ROWW), jnp.float32),
        pltpu.SemaphoreType.DMA,
        pltpu.SemaphoreType.DMA,
        pltpu.SemaphoreType.DMA,
        pltpu.SemaphoreType.DMA,
        pltpu.SemaphoreType.DMA,
        pltpu.SemaphoreType.DMA,
        pltpu.SemaphoreType.DMA,
    ],
    compiler_params=pltpu.CompilerParams(use_tc_tiling_on_sc=False),
)(_sc_body)


def _tc_body(p_ref, x_ref, wl_ref, wr_ref, b_ref, o_ref):
    p = p_ref[0] + p_ref[1]
    cnt = p[:, D:D + 1]
    mean = p[:, :D] / jnp.maximum(cnt, 1.0)
    dn = (((1,), (1,)), ((), ()))
    o_ref[...] = (
        lax.dot_general(mean, wl_ref[...], dn,
                        preferred_element_type=jnp.float32)
        + lax.dot_general(x_ref[...], wr_ref[...], dn,
                          preferred_element_type=jnp.float32)
        + b_ref[...]
    )


def _tc_combine(partial, x, wl_t, wr_t, b2):
    R = 2000
    grid = (N // R,)
    return pl.pallas_call(
        _tc_body,
        grid=grid,
        in_specs=[
            pl.BlockSpec((NC, R, ROWW), lambda i: (0, i, 0)),
            pl.BlockSpec((R, D), lambda i: (i, 0)),
            pl.BlockSpec((D, D), lambda i: (0, 0)),
            pl.BlockSpec((D, D), lambda i: (0, 0)),
            pl.BlockSpec((1, D), lambda i: (0, 0)),
        ],
        out_specs=pl.BlockSpec((R, D), lambda i: (i, 0)),
        out_shape=jax.ShapeDtypeStruct((N, D), jnp.float32),
    )(partial, x, wl_t, wr_t, b2)


def kernel(x, edge_index, W_l, b_l, W_r):
    xpad = jnp.concatenate([x, jnp.ones((N, ROWW - D), jnp.float32)], axis=1)
    zeros = jnp.zeros((ZPT, ROWW), jnp.float32)
    partial = _sc_aggregate(xpad, edge_index, zeros)
    partial = partial.reshape(NC, N, ROWW)
    return _tc_combine(partial, x, W_l, W_r, b_l.reshape(1, D))


# speedup vs baseline: 1.0342x; 1.0014x over previous
import functools

import jax
import jax.numpy as jnp
from jax import lax
from jax.experimental import pallas as pl
from jax.experimental.pallas import tpu as pltpu
from jax.experimental.pallas import tpu_sc as plsc

N = 10000
NPAD = 10240
E = 320000
D = 128
ROWW = D + 16
NC, NS = 2, 16
NW = NC * NS
CH = 256
NCHUNKS = E // CH
SLOTS = 40
ZPT = NPAD // NS
DPT = N // NS


def _sc_body(xpad_hbm, ei_hbm, zeros_hbm, out_hbm,
             eidx0, eidx1, rows, acc,
             si0, si1, sg0, sg1, ss0, ss1, sz):
    c = lax.axis_index("c")
    s = lax.axis_index("s")
    wid = c * NS + s

    zslice = acc.at[pl.ds(pl.multiple_of(s * ZPT, 8), ZPT)]
    pltpu.async_copy(zeros_hbm, zslice, sz)

    def off(t):
        return jnp.minimum(wid + t * NW, NCHUNKS - 1) * CH

    def idx_start(t, ebuf, sem):
        pltpu.async_copy(ei_hbm.at[:, pl.ds(off(t), CH)], ebuf, sem)

    def idx_wait(ebuf, sem):
        pltpu.make_async_copy(ei_hbm.at[:, pl.ds(0, CH)], ebuf, sem).wait()

    H = CH // 2

    def gath(ebuf):
        pltpu.async_copy(xpad_hbm.at[ebuf.at[0, pl.ds(0, H)]],
                         rows.at[pl.ds(0, H)], sg0)
        pltpu.async_copy(xpad_hbm.at[ebuf.at[0, pl.ds(H, H)]],
                         rows.at[pl.ds(H, H)], sg1)

    def gwait(ebuf):
        pltpu.make_async_copy(xpad_hbm.at[ebuf.at[0, pl.ds(0, H)]],
                              rows.at[pl.ds(0, H)], sg0).wait()
        pltpu.make_async_copy(xpad_hbm.at[ebuf.at[0, pl.ds(H, H)]],
                              rows.at[pl.ds(H, H)], sg1).wait()

    def scat(t, ebuf):
        @pl.when(wid + t * NW < NCHUNKS)
        def _():
            pltpu.async_copy(rows.at[pl.ds(0, H)],
                             acc.at[ebuf.at[1, pl.ds(0, H)]], ss0, add=True)
            pltpu.async_copy(rows.at[pl.ds(H, H)],
                             acc.at[ebuf.at[1, pl.ds(H, H)]], ss1, add=True)
            pltpu.make_async_copy(rows.at[pl.ds(0, H)],
                                  acc.at[ebuf.at[1, pl.ds(0, H)]], ss0).wait()
            pltpu.make_async_copy(rows.at[pl.ds(H, H)],
                                  acc.at[ebuf.at[1, pl.ds(H, H)]], ss1).wait()

    idx_start(0, eidx0, si0)
    idx_start(1, eidx1, si1)
    idx_wait(eidx0, si0)
    gath(eidx0)
    pltpu.make_async_copy(zeros_hbm, zslice, sz).wait()
    plsc.subcore_barrier()

    def body(i, carry):
        a = 2 * i
        b = a + 1
        gwait(eidx0)
        scat(a, eidx0)
        idx_start(a + 2, eidx0, si0)
        idx_wait(eidx1, si1)
        gath(eidx1)
        gwait(eidx1)
        scat(b, eidx1)
        idx_start(b + 2, eidx1, si1)
        idx_wait(eidx0, si0)
        gath(eidx0)
        return carry

    lax.fori_loop(0, SLOTS // 2, body, 0)
    idx_wait(eidx1, si1)
    gwait(eidx0)

    plsc.subcore_barrier()
    pltpu.sync_copy(acc.at[pl.ds(s * DPT, DPT)],
                    out_hbm.at[pl.ds(c * N + s * DPT, DPT)])


_sc_aggregate = functools.partial(
    pl.kernel,
    out_type=jax.ShapeDtypeStruct((NC * N, ROWW), jnp.float32),
    mesh=plsc.VectorSubcoreMesh(core_axis_name="c", subcore_axis_name="s",
                                num_cores=NC, num_subcores=NS),
    scratch_types=[
        pltpu.VMEM((2, CH), jnp.int32),
        pltpu.VMEM((2, CH), jnp.int32),
        pltpu.VMEM((CH, ROWW), jnp.float32),
        pltpu.VMEM_SHARED((NPAD, ROWW), jnp.float32),
        pltpu.SemaphoreType.DMA,
        pltpu.SemaphoreType.DMA,
        pltpu.SemaphoreType.DMA,
        pltpu.SemaphoreType.DMA,
        pltpu.SemaphoreType.DMA,
        pltpu.SemaphoreType.DMA,
        pltpu.SemaphoreType.DMA,
    ],
    compiler_params=pltpu.CompilerParams(use_tc_tiling_on_sc=False),
)(_sc_body)


def _tc_body(p_ref, x_ref, wl_ref, wr_ref, b_ref, o_ref):
    p = p_ref[0] + p_ref[1]
    cnt = p[:, D:D + 1]
    mean = p[:, :D] / jnp.maximum(cnt, 1.0)
    dn = (((1,), (1,)), ((), ()))
    o_ref[...] = (
        lax.dot_general(mean, wl_ref[...], dn,
                        preferred_element_type=jnp.float32)
        + lax.dot_general(x_ref[...], wr_ref[...], dn,
                          preferred_element_type=jnp.float32)
        + b_ref[...]
    )


def _tc_combine(partial, x, w_l, w_r, b2):
    R = 2000
    grid = (N // R,)
    return pl.pallas_call(
        _tc_body,
        grid=grid,
        in_specs=[
            pl.BlockSpec((NC, R, ROWW), lambda i: (0, i, 0)),
            pl.BlockSpec((R, D), lambda i: (i, 0)),
            pl.BlockSpec((D, D), lambda i: (0, 0)),
            pl.BlockSpec((D, D), lambda i: (0, 0)),
            pl.BlockSpec((1, D), lambda i: (0, 0)),
        ],
        out_specs=pl.BlockSpec((R, D), lambda i: (i, 0)),
        out_shape=jax.ShapeDtypeStruct((N, D), jnp.float32),
    )(partial, x, w_l, w_r, b2)


def kernel(x, edge_index, W_l, b_l, W_r):
    xpad = jnp.concatenate([x, jnp.ones((N, ROWW - D), jnp.float32)], axis=1)
    zeros = jnp.zeros((ZPT, ROWW), jnp.float32)
    partial = _sc_aggregate(xpad, edge_index, zeros)
    partial = partial.reshape(NC, N, ROWW)
    return _tc_combine(partial, x, W_l, W_r, b_l.reshape(1, D))
